# baseline (speedup 1.0000x reference)
import functools

import jax
import jax.numpy as jnp
from jax import lax
from jax.experimental import pallas as pl
from jax.experimental.pallas import tpu as pltpu
from jax.experimental.pallas import tpu_sc as plsc

BATCH = 4096
HIST = 50
EMBED_DIM = 64
VOCAB = 100000

_NC, _NS = 2, 16
_NW = _NC * _NS
_VT = (VOCAB + 127) // 128
_VT_FULL = VOCAB // 128
_TPW = (_VT + _NW - 1) // _NW
_S = VOCAB // 2

_mesh = plsc.VectorSubcoreMesh(core_axis_name="c", subcore_axis_name="s")


def _wid():
    return lax.axis_index("s") * _NC + lax.axis_index("c")


def _iota16():
    return lax.broadcasted_iota(jnp.int32, (16,), 0)


def _transpose_block(src, dst, ncols):
    dvecs = [d0 * 16 + _iota16() for d0 in range(4)]

    def body(s, _):
        for p in range(2):
            col = jnp.broadcast_to(2 * s + p, (16,))
            for d0 in range(4):
                v = plsc.load_gather(src, [dvecs[d0], col])
                dst[s, pl.ds(p * 64 + d0 * 16, 16)] = v
        return ()

    lax.fori_loop(0, ncols // 2, body, ())


def _table_repack(table_t, tail16, t128, src_v, dst_v, sem):
    wid = _wid()

    def tile_body(j, _):
        vj = j * _NW + wid

        @pl.when(vj < _VT_FULL)
        def _full():
            pltpu.async_copy(
                table_t.at[:, pl.ds(vj * 128, 128)], src_v, sem
            ).wait()
            _transpose_block(src_v, dst_v, 128)
            pltpu.sync_copy(dst_v, t128.at[pl.ds(vj * 64, 64)])

        return ()

    lax.fori_loop(0, _TPW, tile_body, ())

    @pl.when(wid == _NW - 1)
    def _tail():
        pltpu.sync_copy(tail16, src_v.at[pl.ds(0, 16)])
        pltpu.sync_copy(src_v.at[pl.ds(0, 16)], t128.at[pl.ds(_VT_FULL * 64, 16)])


_repack_call = functools.partial(
    pl.kernel,
    mesh=_mesh,
    compiler_params=pltpu.CompilerParams(needs_layout_passes=False),
    out_type=jax.ShapeDtypeStruct((_S, 128), jnp.float32),
    scratch_types=[
        pltpu.VMEM((64, 128), jnp.float32),
        pltpu.VMEM((64, 128), jnp.float32),
        pltpu.SemaphoreType.DMA,
    ],
)(_table_repack)


_BPW = BATCH // _NW


def _gather_t(idx_t, t128, ot, idx_v, s_buf, pb_buf, smp0, smp1, ostg0,
              ostg1, gsem0, gsem1, osem0, osem1):
    wid = _wid()
    b0 = wid * _BPW
    pltpu.sync_copy(idx_t.at[:, pl.ds(b0, _BPW)], idx_v)

    jvecs = [jb * 16 + _iota16() for jb in range(8)]

    def prep(h, _):
        for jb in range(8):
            v = idx_v[h, pl.ds(jb * 16, 16)]
            s_buf[h, pl.ds(jb * 16, 16)] = lax.shift_right_logical(v, 1)
            pb_buf[h, pl.ds(jb * 16, 16)] = lax.shift_left(
                lax.bitwise_and(v, 1), 6
            )
        return ()

    lax.fori_loop(0, HIST, prep, ())

    def gfire(h, smp, sem):
        return pltpu.async_copy(t128.at[s_buf.at[h]], smp, sem)

    def transpose(h, smp, ostg):
        for jb in range(8):
            col0 = pb_buf[h, pl.ds(jb * 16, 16)]
            jv = jvecs[jb]

            def dgrp(g, col, jv=jv, smp=smp, ostg=ostg, jb=jb):
                for dd in range(8):
                    v = plsc.load_gather(smp, [jv, col])
                    ostg[g * 8 + dd, pl.ds(jb * 16, 16)] = v
                    col = col + 1
                return col

            lax.fori_loop(0, 8, dgrp, col0)

    def ofire(h, ostg, sem):
        return pltpu.async_copy(ostg, ot.at[h, :, pl.ds(b0, _BPW)], sem)

    gfire(0, smp0, gsem0)

    @pl.loop(0, HIST, step=2)
    def hloop(h):
        gfire(h + 1, smp1, gsem1)
        pltpu.make_async_copy(t128.at[s_buf.at[h]], smp0, gsem0).wait()

        @pl.when(h >= 2)
        def _():
            pltpu.make_async_copy(
                ostg0, ot.at[h, :, pl.ds(b0, _BPW)], osem0
            ).wait()

        transpose(h, smp0, ostg0)
        ofire(h, ostg0, osem0)

        @pl.when(h + 2 <= HIST - 1)
        def _():
            gfire(h + 2, smp0, gsem0)

        pltpu.make_async_copy(t128.at[s_buf.at[h + 1]], smp1, gsem1).wait()

        @pl.when(h >= 2)
        def _():
            pltpu.make_async_copy(
                ostg1, ot.at[h, :, pl.ds(b0, _BPW)], osem1
            ).wait()

        transpose(h + 1, smp1, ostg1)
        ofire(h + 1, ostg1, osem1)

    pltpu.make_async_copy(ostg0, ot.at[0, :, pl.ds(b0, _BPW)], osem0).wait()
    pltpu.make_async_copy(ostg1, ot.at[0, :, pl.ds(b0, _BPW)], osem1).wait()


_gather_call = functools.partial(
    pl.kernel,
    mesh=_mesh,
    compiler_params=pltpu.CompilerParams(needs_layout_passes=False),
    out_type=jax.ShapeDtypeStruct((HIST, EMBED_DIM, BATCH), jnp.float32),
    scratch_types=[
        pltpu.VMEM((HIST, _BPW), jnp.int32),
        pltpu.VMEM((HIST, _BPW), jnp.int32),
        pltpu.VMEM((HIST, _BPW), jnp.int32),
        pltpu.VMEM((_BPW, 128), jnp.float32),
        pltpu.VMEM((_BPW, 128), jnp.float32),
        pltpu.VMEM((EMBED_DIM, _BPW), jnp.float32),
        pltpu.VMEM((EMBED_DIM, _BPW), jnp.float32),
        pltpu.SemaphoreType.DMA,
        pltpu.SemaphoreType.DMA,
        pltpu.SemaphoreType.DMA,
        pltpu.SemaphoreType.DMA,
    ],
)(_gather_t)


def kernel(inputs, table):
    tail16 = table[_VT_FULL * 128:].reshape(16, 128)
    t128 = _repack_call(table.T, tail16)
    idx_t = inputs.astype(jnp.int32).T
    ot = _gather_call(idx_t, t128)
    return jnp.transpose(ot, (2, 0, 1))


# speedup vs baseline: 2.1255x; 2.1255x over previous
import functools

import jax
import jax.numpy as jnp
from jax import lax
from jax.experimental import pallas as pl
from jax.experimental.pallas import tpu as pltpu
from jax.experimental.pallas import tpu_sc as plsc

BATCH = 4096
HIST = 50
EMBED_DIM = 64

_N = BATCH * HIST
_NC, _NS = 2, 16
_NW = _NC * _NS
_NPW = _N // _NW
_G = 128
_CH = 5
_CHN = _CH * _G
_NCHUNK = _NPW // _CHN


def _sc_gather(idx_hbm, table_hbm, out_hbm, idx_v, rows0, rows1, gsem, osem):
    wid = lax.axis_index("s") * _NC + lax.axis_index("c")
    base = wid * _NPW
    pltpu.sync_copy(idx_hbm.at[pl.ds(base, _NPW)], idx_v)

    bufs = (rows0, rows1)

    def fire(j, buf):
        return [
            pltpu.async_copy(
                table_hbm.at[idx_v.at[pl.ds(j * _CHN + k * _G, _G)]],
                buf.at[pl.ds(k * _G, _G)],
                gsem,
            )
            for k in range(_CH)
        ]

    gathers = {0: fire(0, bufs[0])}
    outs = {}
    for j in range(_NCHUNK):
        b = j % 2
        if j >= 1:
            outs.pop(j - 1).wait()
        if j + 1 < _NCHUNK:
            gathers[j + 1] = fire(j + 1, bufs[1 - b])
        for c in gathers.pop(j):
            c.wait()
        outs[j] = pltpu.async_copy(
            bufs[b], out_hbm.at[pl.ds(base + j * _CHN, _CHN)], osem
        )
    outs.pop(_NCHUNK - 1).wait()


_call = functools.partial(
    pl.kernel,
    mesh=plsc.VectorSubcoreMesh(core_axis_name="c", subcore_axis_name="s"),
    compiler_params=pltpu.CompilerParams(use_tc_tiling_on_sc=False),
    out_type=jax.ShapeDtypeStruct((_N, EMBED_DIM), jnp.float32),
    scratch_types=[
        pltpu.VMEM((_NPW,), jnp.int32),
        pltpu.VMEM((_CHN, EMBED_DIM), jnp.float32),
        pltpu.VMEM((_CHN, EMBED_DIM), jnp.float32),
        pltpu.SemaphoreType.DMA,
        pltpu.SemaphoreType.DMA,
    ],
)(_sc_gather)


def kernel(inputs, table):
    idx = inputs.astype(jnp.int32).reshape(_N)
    out = _call(idx, table)
    return out.reshape(BATCH, HIST, EMBED_DIM)
